# baseline (speedup 1.0000x reference)
import jax
import jax.numpy as jnp
from jax import lax
from jax.experimental import pallas as pl
from jax.experimental.pallas import tpu as pltpu
from jax.experimental.pallas import tpu_sc as plsc

_TEMPERATURE = 0.07
_BASE_TEMPERATURE = 0.07

_B = 128
_M = 16384
_D = 2048
_CHK = 2048
_NCHK = _M // _CHK

_INT_MAX = 2147483647


def _pipeline(logits_hbm, bufs, sems, chunk_fn):
    def start(c):
        return pltpu.make_async_copy(
            logits_hbm.at[:, pl.ds(c * _CHK, _CHK)],
            bufs[c % 2], sems[c % 2])

    start(0).start()
    start(1).start()
    for c in range(_NCHK):
        start(c).wait()
        chunk_fn(c, bufs[c % 2][...])
        if c + 2 < _NCHK:
            start(c + 2).start()


def _argmin_body(logits_hbm, cid_ref, tid_ref, cam_ref, trk_ref, hidx_ref,
                 bufa, bufb, sema, semb):
    keycol = (cam_ref[...] << 16) | trk_ref[...]
    state = {
        "hmin": jnp.full((_B, 1), jnp.inf, jnp.float32),
        "hidx": jnp.zeros((_B, 1), jnp.int32),
    }

    def chunk_fn(c, logits):
        sl = pl.ds(c * _CHK, _CHK)
        pos = ((cid_ref[:, sl] << 16) | tid_ref[:, sl]) == keycol
        v = jnp.where(pos, logits, jnp.inf)
        idx = lax.broadcasted_iota(jnp.int32, v.shape, 1) + c * _CHK
        w = _CHK // 2
        while w >= 128:
            v1, v2 = v[:, :w], v[:, w:]
            i1, i2 = idx[:, :w], idx[:, w:]
            lt = v2 < v1
            eq = v2 == v1
            v = jnp.minimum(v1, v2)
            idx = jnp.where(lt, i2, jnp.where(eq, jnp.minimum(i1, i2), i1))
            w //= 2
        blk_min = jnp.min(v, axis=1, keepdims=True)
        blk_arg = jnp.min(jnp.where(v == blk_min, idx, jnp.int32(_INT_MAX)),
                          axis=1, keepdims=True)
        take = blk_min < state["hmin"]
        state["hidx"] = jnp.where(take, blk_arg, state["hidx"])
        state["hmin"] = jnp.where(take, blk_min, state["hmin"])

    _pipeline(logits_hbm, (bufa, bufb), (sema, semb), chunk_fn)
    hidx_ref[...] = state["hidx"]


_argmin_call = pl.pallas_call(
    _argmin_body,
    in_specs=[
        pl.BlockSpec(memory_space=pl.ANY),
        pl.BlockSpec((1, _M), lambda: (0, 0)),
        pl.BlockSpec((1, _M), lambda: (0, 0)),
        pl.BlockSpec((_B, 1), lambda: (0, 0)),
        pl.BlockSpec((_B, 1), lambda: (0, 0)),
    ],
    out_specs=pl.BlockSpec((_B, 1), lambda: (0, 0)),
    out_shape=jax.ShapeDtypeStruct((_B, 1), jnp.int32),
    scratch_shapes=[
        pltpu.VMEM((_B, _CHK), jnp.float32),
        pltpu.VMEM((_B, _CHK), jnp.float32),
        pltpu.SemaphoreType.DMA,
        pltpu.SemaphoreType.DMA,
    ],
)


_NC = 1
_NS = 16
_NW = _NC * _NS
_RPW = _B // _NW


def _gather_body(mem_hbm, idx_hbm, out_hbm, idx_v, rows_v, sem):
    wid = lax.axis_index("s") * _NC + lax.axis_index("c")
    base = wid * _RPW
    pltpu.sync_copy(idx_hbm.at[pl.ds(base, _RPW)], idx_v)
    pltpu.async_copy(mem_hbm.at[idx_v], rows_v, sem).wait()
    pltpu.sync_copy(rows_v, out_hbm.at[pl.ds(base, _RPW)])


_gather_call = pl.kernel(
    _gather_body,
    out_type=jax.ShapeDtypeStruct((_B, _D), jnp.float32),
    mesh=plsc.VectorSubcoreMesh(core_axis_name="c", subcore_axis_name="s",
                                num_cores=_NC),
    scratch_types=[
        pltpu.VMEM((_RPW,), jnp.int32),
        pltpu.VMEM((_RPW, _D), jnp.float32),
        pltpu.SemaphoreType.DMA,
    ],
)


def _loss_body(logits_hbm, cid_ref, tid_ref, cam_ref, trk_ref, loss_ref,
               bufa, bufb, sema, semb):
    keycol = (cam_ref[...] << 16) | trk_ref[...]
    inv_t = jnp.float32(1.0 / _TEMPERATURE)
    st = {
        "m": jnp.full((_B, 1), -jnp.inf, jnp.float32),
        "s": jnp.zeros((_B, 1), jnp.float32),
        "ps": jnp.zeros((_B, 1), jnp.float32),
        "np": jnp.zeros((_B, 1), jnp.float32),
    }

    def chunk_fn(c, logits):
        sl = pl.ds(c * _CHK, _CHK)
        cid = cid_ref[:, sl]
        cam = cid == cam_ref[...]
        pos = ((cid << 16) | tid_ref[:, sl]) == keycol
        a = logits * inv_t
        blk_max = jnp.max(jnp.where(cam, a, -jnp.inf), axis=1, keepdims=True)
        m_new = jnp.maximum(st["m"], blk_max)
        scale = jnp.where(st["m"] == m_new, jnp.float32(1.0),
                          jnp.exp(st["m"] - m_new))
        blk_sum = jnp.sum(jnp.where(cam, jnp.exp(a - m_new), 0.0),
                          axis=1, keepdims=True)
        st["s"] = st["s"] * scale + blk_sum
        st["m"] = m_new
        st["ps"] = st["ps"] + jnp.sum(jnp.where(pos, a, 0.0),
                                      axis=1, keepdims=True)
        st["np"] = st["np"] + jnp.sum(jnp.where(pos, 1.0, 0.0),
                                      axis=1, keepdims=True)

    _pipeline(logits_hbm, (bufa, bufb), (sema, semb), chunk_fn)
    mean_lp = st["ps"] / st["np"] - (st["m"] + jnp.log(st["s"]))
    loss_i = -(_TEMPERATURE / _BASE_TEMPERATURE) * mean_lp
    loss_ref[...] = jnp.sum(loss_i, axis=0, keepdims=True) * jnp.float32(1.0 / _B)


_loss_call = pl.pallas_call(
    _loss_body,
    in_specs=[
        pl.BlockSpec(memory_space=pl.ANY),
        pl.BlockSpec((1, _M), lambda: (0, 0)),
        pl.BlockSpec((1, _M), lambda: (0, 0)),
        pl.BlockSpec((_B, 1), lambda: (0, 0)),
        pl.BlockSpec((_B, 1), lambda: (0, 0)),
    ],
    out_specs=pl.BlockSpec((1, 1), lambda: (0, 0)),
    out_shape=jax.ShapeDtypeStruct((1, 1), jnp.float32),
    scratch_shapes=[
        pltpu.VMEM((_B, _CHK), jnp.float32),
        pltpu.VMEM((_B, _CHK), jnp.float32),
        pltpu.SemaphoreType.DMA,
        pltpu.SemaphoreType.DMA,
    ],
)


def kernel(mem, logits, mem_CID, mem_TID, camids, trackids):
    cid2 = mem_CID.reshape(1, _M)
    tid2 = mem_TID.reshape(1, _M)
    cam2 = camids.reshape(_B, 1)
    trk2 = trackids.reshape(_B, 1)
    hidx2 = _argmin_call(logits, cid2, tid2, cam2, trk2)
    hard_pos = _gather_call(mem, hidx2.reshape(_B))
    loss2 = _loss_call(logits, cid2, tid2, cam2, trk2)
    return loss2[0, 0], hard_pos


# speedup vs baseline: 1.0488x; 1.0488x over previous
import jax
import jax.numpy as jnp
from jax import lax
from jax.experimental import pallas as pl
from jax.experimental.pallas import tpu as pltpu
from jax.experimental.pallas import tpu_sc as plsc

_TEMPERATURE = 0.07
_BASE_TEMPERATURE = 0.07

_B = 128
_M = 16384
_D = 2048
_CHK = 2048
_NCHK = _M // _CHK

_INT_MAX = 2147483647


def _stats_body(logits_hbm, cid_ref, tid_ref, cam_ref, trk_ref,
                loss_ref, hidx_ref, bufa, bufb, sema, semb):
    bufs = (bufa, bufb)
    sems = (sema, semb)

    def start(c):
        return pltpu.make_async_copy(
            logits_hbm.at[:, pl.ds(c * _CHK, _CHK)],
            bufs[c % 2], sems[c % 2])

    start(0).start()
    start(1).start()

    keycol = (cam_ref[...] << 16) | trk_ref[...]

    inv_t = jnp.float32(1.0 / _TEMPERATURE)
    ones_col = jnp.ones((_CHK, 1), jnp.float32)
    m_run = jnp.full((_B, 1), -1e30, jnp.float32)
    s_run = jnp.zeros((_B, 1), jnp.float32)
    ps_run = jnp.zeros((_B, 1), jnp.float32)
    np_run = jnp.zeros((_B, 1), jnp.float32)
    hmin = jnp.full((_B, 1), jnp.inf, jnp.float32)
    hidx = jnp.zeros((_B, 1), jnp.int32)

    for c in range(_NCHK):
        start(c).wait()
        logits = bufs[c % 2][...]
        sl = pl.ds(c * _CHK, _CHK)
        cid = cid_ref[:, sl]
        cam = cid == cam_ref[...]
        pos = ((cid << 16) | tid_ref[:, sl]) == keycol

        a = logits * inv_t

        blk_max = jnp.max(a, axis=1, keepdims=True)
        m_new = jnp.maximum(m_run, blk_max)
        scale = jnp.exp(m_run - m_new)
        camf = jnp.where(cam, 1.0, 0.0)
        posf = jnp.where(pos, 1.0, 0.0)
        expv = jnp.exp(a - m_new) * camf
        blk_sum = lax.dot_general(expv, ones_col, (((1,), (0,)), ((), ())),
                                  preferred_element_type=jnp.float32)
        s_run = s_run * scale + blk_sum
        m_run = m_new

        ps_run = ps_run + lax.dot_general(a * posf, ones_col,
                                          (((1,), (0,)), ((), ())),
                                          preferred_element_type=jnp.float32)
        np_run = np_run + lax.dot_general(posf, ones_col,
                                          (((1,), (0,)), ((), ())),
                                          preferred_element_type=jnp.float32)

        v = jnp.where(pos, a, jnp.inf)
        idx = lax.broadcasted_iota(jnp.int32, v.shape, 1) + c * _CHK
        w = _CHK // 2
        while w >= 128:
            v1, v2 = v[:, :w], v[:, w:]
            i1, i2 = idx[:, :w], idx[:, w:]
            lt = v2 < v1
            eq = v2 == v1
            v = jnp.minimum(v1, v2)
            idx = jnp.where(lt, i2, jnp.where(eq, jnp.minimum(i1, i2), i1))
            w //= 2
        blk_min = jnp.min(v, axis=1, keepdims=True)
        blk_arg = jnp.min(jnp.where(v == blk_min, idx, jnp.int32(_INT_MAX)),
                          axis=1, keepdims=True)
        take = blk_min < hmin
        hidx = jnp.where(take, blk_arg, hidx)
        hmin = jnp.where(take, blk_min, hmin)

        if c + 2 < _NCHK:
            start(c + 2).start()

    mean_lp = ps_run / np_run - (m_run + jnp.log(s_run))
    loss_i = -(_TEMPERATURE / _BASE_TEMPERATURE) * mean_lp
    loss_ref[...] = jnp.sum(loss_i, axis=0, keepdims=True) * jnp.float32(1.0 / _B)
    hidx_ref[...] = hidx


_stats_call = pl.pallas_call(
    _stats_body,
    in_specs=[
        pl.BlockSpec(memory_space=pl.ANY),
        pl.BlockSpec((1, _M), lambda: (0, 0)),
        pl.BlockSpec((1, _M), lambda: (0, 0)),
        pl.BlockSpec((_B, 1), lambda: (0, 0)),
        pl.BlockSpec((_B, 1), lambda: (0, 0)),
    ],
    out_specs=[
        pl.BlockSpec((1, 1), lambda: (0, 0)),
        pl.BlockSpec((_B, 1), lambda: (0, 0)),
    ],
    out_shape=[
        jax.ShapeDtypeStruct((1, 1), jnp.float32),
        jax.ShapeDtypeStruct((_B, 1), jnp.int32),
    ],
    scratch_shapes=[
        pltpu.VMEM((_B, _CHK), jnp.float32),
        pltpu.VMEM((_B, _CHK), jnp.float32),
        pltpu.SemaphoreType.DMA,
        pltpu.SemaphoreType.DMA,
    ],
)


_NC = 1
_NS = 16
_NW = _NC * _NS
_RPW = _B // _NW


def _gather_body(mem_hbm, idx_hbm, out_hbm, idx_v, rows_v, sem):
    wid = lax.axis_index("s") * _NC + lax.axis_index("c")
    base = wid * _RPW
    pltpu.sync_copy(idx_hbm.at[pl.ds(base, _RPW)], idx_v)
    pltpu.async_copy(mem_hbm.at[idx_v], rows_v, sem).wait()
    pltpu.sync_copy(rows_v, out_hbm.at[pl.ds(base, _RPW)])


_gather_call = pl.kernel(
    _gather_body,
    out_type=jax.ShapeDtypeStruct((_B, _D), jnp.float32),
    mesh=plsc.VectorSubcoreMesh(core_axis_name="c", subcore_axis_name="s",
                                num_cores=_NC),
    scratch_types=[
        pltpu.VMEM((_RPW,), jnp.int32),
        pltpu.VMEM((_RPW, _D), jnp.float32),
        pltpu.SemaphoreType.DMA,
    ],
)


def kernel(mem, logits, mem_CID, mem_TID, camids, trackids):
    loss2, hidx2 = _stats_call(
        logits,
        mem_CID.reshape(1, _M),
        mem_TID.reshape(1, _M),
        camids.reshape(_B, 1),
        trackids.reshape(_B, 1),
    )
    hard_pos = _gather_call(mem, hidx2.reshape(_B))
    return loss2[0, 0], hard_pos


# speedup vs baseline: 1.0798x; 1.0296x over previous
import jax
import jax.numpy as jnp
from jax import lax
from jax.experimental import pallas as pl
from jax.experimental.pallas import tpu as pltpu
from jax.experimental.pallas import tpu_sc as plsc

_TEMPERATURE = 0.07
_BASE_TEMPERATURE = 0.07

_B = 128
_M = 16384
_D = 2048
_CHK = 4096
_NCHK = _M // _CHK

_INT_MAX = 2147483647


def _stats_body(logits_hbm, cid_ref, tid_ref, cam_ref, trk_ref,
                loss_ref, hidx_ref, bufa, bufb, sema, semb):
    bufs = (bufa, bufb)
    sems = (sema, semb)

    def start(c):
        return pltpu.make_async_copy(
            logits_hbm.at[:, pl.ds(c * _CHK, _CHK)],
            bufs[c % 2], sems[c % 2])

    start(0).start()
    start(1).start()

    keycol = (cam_ref[...] << 16) | trk_ref[...]

    inv_t = jnp.float32(1.0 / _TEMPERATURE)
    m_run = jnp.full((_B, 1), -jnp.inf, jnp.float32)
    s_run = jnp.zeros((_B, 1), jnp.float32)
    ps_run = jnp.zeros((_B, 1), jnp.float32)
    np_run = jnp.zeros((_B, 1), jnp.float32)
    hmin = jnp.full((_B, 1), jnp.inf, jnp.float32)
    hidx = jnp.zeros((_B, 1), jnp.int32)

    for c in range(_NCHK):
        start(c).wait()
        logits = bufs[c % 2][...]
        sl = pl.ds(c * _CHK, _CHK)
        cid = cid_ref[:, sl]
        cam = cid == cam_ref[...]
        pos = ((cid << 16) | tid_ref[:, sl]) == keycol

        a = logits * inv_t

        blk_max = jnp.max(jnp.where(cam, a, -jnp.inf), axis=1, keepdims=True)
        m_new = jnp.maximum(m_run, blk_max)
        scale = jnp.where(m_run == m_new, jnp.float32(1.0),
                          jnp.exp(m_run - m_new))
        blk_sum = jnp.sum(jnp.where(cam, jnp.exp(a - m_new), 0.0),
                          axis=1, keepdims=True)
        s_run = s_run * scale + blk_sum
        m_run = m_new

        ps_run = ps_run + jnp.sum(jnp.where(pos, a, 0.0),
                                  axis=1, keepdims=True)
        np_run = np_run + jnp.sum(jnp.where(pos, 1.0, 0.0),
                                  axis=1, keepdims=True)

        v = jnp.where(pos, a, jnp.inf)
        idx = lax.broadcasted_iota(jnp.int32, v.shape, 1) + c * _CHK
        w = _CHK // 2
        while w >= 128:
            v1, v2 = v[:, :w], v[:, w:]
            i1, i2 = idx[:, :w], idx[:, w:]
            lt = v2 < v1
            eq = v2 == v1
            v = jnp.minimum(v1, v2)
            idx = jnp.where(lt, i2, jnp.where(eq, jnp.minimum(i1, i2), i1))
            w //= 2
        blk_min = jnp.min(v, axis=1, keepdims=True)
        blk_arg = jnp.min(jnp.where(v == blk_min, idx, jnp.int32(_INT_MAX)),
                          axis=1, keepdims=True)
        take = blk_min < hmin
        hidx = jnp.where(take, blk_arg, hidx)
        hmin = jnp.where(take, blk_min, hmin)

        if c + 2 < _NCHK:
            start(c + 2).start()

    mean_lp = ps_run / np_run - (m_run + jnp.log(s_run))
    loss_i = -(_TEMPERATURE / _BASE_TEMPERATURE) * mean_lp
    loss_ref[...] = jnp.sum(loss_i, axis=0, keepdims=True) * jnp.float32(1.0 / _B)
    hidx_ref[...] = hidx


_stats_call = pl.pallas_call(
    _stats_body,
    in_specs=[
        pl.BlockSpec(memory_space=pl.ANY),
        pl.BlockSpec((1, _M), lambda: (0, 0)),
        pl.BlockSpec((1, _M), lambda: (0, 0)),
        pl.BlockSpec((_B, 1), lambda: (0, 0)),
        pl.BlockSpec((_B, 1), lambda: (0, 0)),
    ],
    out_specs=[
        pl.BlockSpec((1, 1), lambda: (0, 0)),
        pl.BlockSpec((_B, 1), lambda: (0, 0)),
    ],
    out_shape=[
        jax.ShapeDtypeStruct((1, 1), jnp.float32),
        jax.ShapeDtypeStruct((_B, 1), jnp.int32),
    ],
    scratch_shapes=[
        pltpu.VMEM((_B, _CHK), jnp.float32),
        pltpu.VMEM((_B, _CHK), jnp.float32),
        pltpu.SemaphoreType.DMA,
        pltpu.SemaphoreType.DMA,
    ],
)


_NC = 1
_NS = 16
_NW = _NC * _NS
_RPW = _B // _NW


def _gather_body(mem_hbm, idx_hbm, out_hbm, idx_v, rows_v, sem):
    wid = lax.axis_index("s") * _NC + lax.axis_index("c")
    base = wid * _RPW
    pltpu.sync_copy(idx_hbm.at[pl.ds(base, _RPW)], idx_v)
    pltpu.async_copy(mem_hbm.at[idx_v], rows_v, sem).wait()
    pltpu.sync_copy(rows_v, out_hbm.at[pl.ds(base, _RPW)])


_gather_call = pl.kernel(
    _gather_body,
    out_type=jax.ShapeDtypeStruct((_B, _D), jnp.float32),
    mesh=plsc.VectorSubcoreMesh(core_axis_name="c", subcore_axis_name="s",
                                num_cores=_NC),
    scratch_types=[
        pltpu.VMEM((_RPW,), jnp.int32),
        pltpu.VMEM((_RPW, _D), jnp.float32),
        pltpu.SemaphoreType.DMA,
    ],
)


def kernel(mem, logits, mem_CID, mem_TID, camids, trackids):
    loss2, hidx2 = _stats_call(
        logits,
        mem_CID.reshape(1, _M),
        mem_TID.reshape(1, _M),
        camids.reshape(_B, 1),
        trackids.reshape(_B, 1),
    )
    hard_pos = _gather_call(mem, hidx2.reshape(_B))
    return loss2[0, 0], hard_pos


# speedup vs baseline: 1.0988x; 1.0176x over previous
import jax
import jax.numpy as jnp
from jax import lax
from jax.experimental import pallas as pl
from jax.experimental.pallas import tpu as pltpu
from jax.experimental.pallas import tpu_sc as plsc

_TEMPERATURE = 0.07
_BASE_TEMPERATURE = 0.07

_B = 128
_M = 16384
_D = 2048
_BLK = 4096
_NBLK = _M // _BLK

_INT_MAX = 2147483647


def _stats_body(logits_ref, cid_ref, tid_ref, cam_ref, trk_ref,
                loss_ref, hidx_ref,
                m_scr, s_scr, ps_scr, np_scr, hmin_scr, hidx_scr):
    j = pl.program_id(0)

    @pl.when(j == 0)
    def _init():
        m_scr[...] = jnp.full(m_scr.shape, -jnp.inf, m_scr.dtype)
        s_scr[...] = jnp.zeros(s_scr.shape, s_scr.dtype)
        ps_scr[...] = jnp.zeros(ps_scr.shape, ps_scr.dtype)
        np_scr[...] = jnp.zeros(np_scr.shape, np_scr.dtype)
        hmin_scr[...] = jnp.full(hmin_scr.shape, jnp.inf, hmin_scr.dtype)
        hidx_scr[...] = jnp.zeros(hidx_scr.shape, hidx_scr.dtype)

    logits = logits_ref[...]
    cid = cid_ref[...]
    cam = cid == cam_ref[...]
    keyrow = (cid << 16) | tid_ref[...]
    keycol = (cam_ref[...] << 16) | trk_ref[...]
    pos = keyrow == keycol

    a = logits * jnp.float32(1.0 / _TEMPERATURE)

    blk_max = jnp.max(jnp.where(cam, a, -jnp.inf), axis=1, keepdims=True)
    m_old = m_scr[...]
    m_new = jnp.maximum(m_old, blk_max)
    scale = jnp.where(m_old == m_new, jnp.float32(1.0), jnp.exp(m_old - m_new))
    blk_sum = jnp.sum(jnp.where(cam, jnp.exp(a - m_new), 0.0),
                      axis=1, keepdims=True)
    s_scr[...] = s_scr[...] * scale + blk_sum
    m_scr[...] = m_new

    ps_scr[...] = ps_scr[...] + jnp.sum(jnp.where(pos, a, 0.0),
                                        axis=1, keepdims=True)
    np_scr[...] = np_scr[...] + jnp.sum(jnp.where(pos, 1.0, 0.0),
                                        axis=1, keepdims=True)

    v = jnp.where(pos, a, jnp.inf)
    idx = lax.broadcasted_iota(jnp.int32, v.shape, 1) + j * _BLK
    w = _BLK // 2
    while w >= 128:
        v1, v2 = v[:, :w], v[:, w:]
        i1, i2 = idx[:, :w], idx[:, w:]
        lt = v2 < v1
        eq = v2 == v1
        v = jnp.minimum(v1, v2)
        idx = jnp.where(lt, i2, jnp.where(eq, jnp.minimum(i1, i2), i1))
        w //= 2
    blk_min = jnp.min(v, axis=1, keepdims=True)
    blk_arg = jnp.min(jnp.where(v == blk_min, idx, jnp.int32(_INT_MAX)),
                      axis=1, keepdims=True)
    better = blk_min < hmin_scr[...]
    tie = jnp.logical_and(blk_min == hmin_scr[...], blk_arg < hidx_scr[...])
    upd = jnp.logical_or(better, tie)
    hidx_scr[...] = jnp.where(upd, blk_arg, hidx_scr[...])
    hmin_scr[...] = jnp.where(better, blk_min, hmin_scr[...])

    @pl.when(j == _NBLK - 1)
    def _fin():
        mean_lp = ps_scr[...] / np_scr[...] - (m_scr[...] + jnp.log(s_scr[...]))
        loss_i = -(_TEMPERATURE / _BASE_TEMPERATURE) * mean_lp
        loss_ref[...] = jnp.sum(loss_i, axis=0, keepdims=True) * jnp.float32(1.0 / _B)
        hidx_ref[...] = hidx_scr[...]


_stats_call = pl.pallas_call(
    _stats_body,
    grid=(_NBLK,),
    in_specs=[
        pl.BlockSpec((_B, _BLK), lambda j: (0, j)),
        pl.BlockSpec((1, _BLK), lambda j: (0, j)),
        pl.BlockSpec((1, _BLK), lambda j: (0, j)),
        pl.BlockSpec((_B, 1), lambda j: (0, 0)),
        pl.BlockSpec((_B, 1), lambda j: (0, 0)),
    ],
    out_specs=[
        pl.BlockSpec((1, 1), lambda j: (0, 0)),
        pl.BlockSpec((_B, 1), lambda j: (0, 0)),
    ],
    out_shape=[
        jax.ShapeDtypeStruct((1, 1), jnp.float32),
        jax.ShapeDtypeStruct((_B, 1), jnp.int32),
    ],
    scratch_shapes=[
        pltpu.VMEM((_B, 1), jnp.float32),
        pltpu.VMEM((_B, 1), jnp.float32),
        pltpu.VMEM((_B, 1), jnp.float32),
        pltpu.VMEM((_B, 1), jnp.float32),
        pltpu.VMEM((_B, 1), jnp.float32),
        pltpu.VMEM((_B, 1), jnp.int32),
    ],
)

_NC = 1
_NS = 16
_NW = _NC * _NS
_RPW = _B // _NW


def _gather_body(mem_hbm, idx_hbm, out_hbm, idx_v, rows_v, sem):
    wid = lax.axis_index("s") * _NC + lax.axis_index("c")
    base = wid * _RPW
    pltpu.sync_copy(idx_hbm.at[pl.ds(base, _RPW)], idx_v)
    pltpu.async_copy(mem_hbm.at[idx_v], rows_v, sem).wait()
    pltpu.sync_copy(rows_v, out_hbm.at[pl.ds(base, _RPW)])


_gather_call = pl.kernel(
    _gather_body,
    out_type=jax.ShapeDtypeStruct((_B, _D), jnp.float32),
    mesh=plsc.VectorSubcoreMesh(core_axis_name="c", subcore_axis_name="s",
                                num_cores=_NC),
    scratch_types=[
        pltpu.VMEM((_RPW,), jnp.int32),
        pltpu.VMEM((_RPW, _D), jnp.float32),
        pltpu.SemaphoreType.DMA,
    ],
)


def kernel(mem, logits, mem_CID, mem_TID, camids, trackids):
    loss2, hidx2 = _stats_call(
        logits,
        mem_CID.reshape(1, _M),
        mem_TID.reshape(1, _M),
        camids.reshape(_B, 1),
        trackids.reshape(_B, 1),
    )
    hard_pos = _gather_call(mem, hidx2.reshape(_B))
    return loss2[0, 0], hard_pos
